# baseline (speedup 1.0000x reference)
import functools

import jax
import jax.numpy as jnp
from jax import lax
from jax.experimental import pallas as pl
from jax.experimental.pallas import tpu as pltpu
from jax.experimental.pallas import tpu_sc as plsc

NC = 2
NS = 16
NW = NC * NS

CHUNK = 128
L = 16

N_CHUNKS_C0 = 152
N_CHUNKS_C1 = 8


def _sc_scatter_kernel(n_pad, n0, n1, d):
    mesh = plsc.VectorSubcoreMesh(core_axis_name="c", subcore_axis_name="s")
    z_rows = n_pad // NS
    n_max = max(n0, n1)

    @functools.partial(
        pl.kernel,
        out_type=(
            jax.ShapeDtypeStruct((n_pad, d), jnp.float32),
            jax.ShapeDtypeStruct((n_pad, d), jnp.float32),
        ),
        mesh=mesh,
        scratch_types=[
            pltpu.VMEM((n_max, CHUNK), jnp.int32),
            pltpu.VMEM((CHUNK,), jnp.int32),
            pltpu.VMEM((CHUNK,), jnp.int32),
            pltpu.VMEM((CHUNK, d), jnp.float32),
            pltpu.VMEM_SHARED((n_pad, d), jnp.float32),
            pltpu.SemaphoreType.DMA,
        ],
    )
    def sc_kernel(x_hbm, idx_hbm, zeros_hbm, out0, out1,
                  idx_v, src_b, dst_b, rows_v, agg_sh, gsem):
        c = lax.axis_index("c")
        s = lax.axis_index("s")

        start = pl.multiple_of(jnp.where(c == 0, s * n0, NS * n0 + s * n1), 8)
        count = jnp.where(c == 0, n0, n1)
        pltpu.sync_copy(idx_hbm.at[pl.ds(start, n_max)], idx_v)

        zslice = pl.ds(s * z_rows, z_rows)
        pltpu.sync_copy(zeros_hbm.at[zslice], agg_sh.at[zslice])
        plsc.subcore_barrier()

        def body(j, carry):
            for k in range(CHUNK // L):
                v = idx_v[j, pl.ds(k * L, L)]
                src_b[pl.ds(k * L, L)] = v & 0xFFFF
                dst_b[pl.ds(k * L, L)] = lax.shift_right_logical(v, 16)
            pltpu.async_copy(x_hbm.at[src_b], rows_v, gsem).wait()
            pltpu.sync_copy(rows_v, agg_sh.at[dst_b], add=True)
            return carry

        lax.fori_loop(0, count, body, 0, unroll=False)
        plsc.subcore_barrier()

        @pl.when(c == 0)
        def _():
            pltpu.sync_copy(agg_sh.at[zslice], out0.at[zslice])

        @pl.when(c == 1)
        def _():
            pltpu.sync_copy(agg_sh.at[zslice], out1.at[zslice])

    return sc_kernel


def _tc_layer_kernel(a0, a1, x, w_rel_t, w_root_t, b_row):
    n, d = x.shape
    blk = 2000
    grid = (n // blk,)

    def body(a0_ref, a1_ref, x_ref, wr_ref, wo_ref, b_ref, o_ref):
        agg = a0_ref[...] + a1_ref[...]
        acc = jnp.dot(agg, wr_ref[...], preferred_element_type=jnp.float32)
        acc += jnp.dot(x_ref[...], wo_ref[...], preferred_element_type=jnp.float32)
        o_ref[...] = jnp.maximum(acc + b_ref[...], 0.0)

    row_spec = pl.BlockSpec((blk, d), lambda i: (i, 0))
    full_spec = pl.BlockSpec((d, d), lambda i: (0, 0))
    bias_spec = pl.BlockSpec((1, d), lambda i: (0, 0))
    return pl.pallas_call(
        body,
        grid=grid,
        in_specs=[row_spec, row_spec, row_spec, full_spec, full_spec, bias_spec],
        out_specs=row_spec,
        out_shape=jax.ShapeDtypeStruct((n, d), jnp.float32),
    )(a0, a1, x, w_rel_t, w_root_t, b_row)


def kernel(x, edge_index, dropout, W1_rel, b1_rel, W1_root, W2_rel, b2_rel, W2_root):
    n, d = x.shape
    e = edge_index.shape[1]

    n0, n1 = N_CHUNKS_C0, N_CHUNKS_C1
    e_pad = NS * (n0 + n1) * CHUNK
    n_pad = 128 * (-(-(n + 1) // 128))
    assert n < 2**15 and e_pad >= e and n0 % 8 == 0 and n1 % 8 == 0

    packed = edge_index[0] | (edge_index[1] << 16)
    pad = e_pad - e
    if pad:
        packed = jnp.concatenate(
            [packed, jnp.full((pad,), n << 16, jnp.int32)])
    packed = packed.reshape(NS * (n0 + n1), CHUNK)
    zeros = jnp.zeros((n_pad, d), jnp.float32)

    sc_scatter = _sc_scatter_kernel(n_pad, n0, n1, d)

    a0, a1 = sc_scatter(x, packed, zeros)
    h = _tc_layer_kernel(a0[:n], a1[:n], x, W1_rel.T, W1_root.T,
                         b1_rel.reshape(1, d))
    a0, a1 = sc_scatter(h, packed, zeros)
    out = _tc_layer_kernel(a0[:n], a1[:n], h, W2_rel.T, W2_root.T,
                           b2_rel.reshape(1, d))
    return out


# speedup vs baseline: 1.1429x; 1.1429x over previous
import functools

import jax
import jax.numpy as jnp
from jax import lax
from jax.experimental import pallas as pl
from jax.experimental.pallas import tpu as pltpu
from jax.experimental.pallas import tpu_sc as plsc

NC = 2
NS = 16
NW = NC * NS

CHUNK = 128
L = 16

N_CHUNKS_C0 = 144
N_CHUNKS_C1 = 16


def _sc_scatter_kernel(n_pad, n0, n1, d):
    mesh = plsc.VectorSubcoreMesh(core_axis_name="c", subcore_axis_name="s")
    z_rows = n_pad // NS
    n_max = max(n0, n1)

    @functools.partial(
        pl.kernel,
        out_type=(
            jax.ShapeDtypeStruct((n_pad, d), jnp.float32),
            jax.ShapeDtypeStruct((n_pad, d), jnp.float32),
        ),
        mesh=mesh,
        scratch_types=[
            pltpu.VMEM((n_max, CHUNK), jnp.int32),
            pltpu.VMEM((CHUNK,), jnp.int32),
            pltpu.VMEM((CHUNK,), jnp.int32),
            pltpu.VMEM((CHUNK, d), jnp.float32),
            pltpu.VMEM_SHARED((n_pad, d), jnp.float32),
            pltpu.SemaphoreType.DMA,
        ],
    )
    def sc_kernel(x_hbm, idx_hbm, zeros_hbm, out0, out1,
                  idx_v, src_b, dst_b, rows_v, agg_sh, gsem):
        c = lax.axis_index("c")
        s = lax.axis_index("s")

        start = pl.multiple_of(jnp.where(c == 0, s * n0, NS * n0 + s * n1), 8)
        count = jnp.where(c == 0, n0, n1)
        pltpu.sync_copy(idx_hbm.at[pl.ds(start, n_max)], idx_v)

        zslice = pl.ds(s * z_rows, z_rows)
        pltpu.sync_copy(zeros_hbm.at[zslice], agg_sh.at[zslice])
        plsc.subcore_barrier()

        def body(j, carry):
            for k in range(CHUNK // L):
                v = idx_v[j, pl.ds(k * L, L)]
                src_b[pl.ds(k * L, L)] = v & 0xFFFF
                dst_b[pl.ds(k * L, L)] = lax.shift_right_logical(v, 16)
            pltpu.async_copy(x_hbm.at[src_b], rows_v, gsem).wait()
            pltpu.sync_copy(rows_v, agg_sh.at[dst_b], add=True)
            return carry

        lax.fori_loop(0, count, body, 0, unroll=False)
        plsc.subcore_barrier()

        @pl.when(c == 0)
        def _():
            pltpu.sync_copy(agg_sh.at[zslice], out0.at[zslice])

        @pl.when(c == 1)
        def _():
            pltpu.sync_copy(agg_sh.at[zslice], out1.at[zslice])

    return sc_kernel


def _tc_layer_kernel(a0, a1, x, w_rel_t, w_root_t, b_row):
    n, d = x.shape
    blk = 2000
    grid = (n // blk,)

    def body(a0_ref, a1_ref, x_ref, wr_ref, wo_ref, b_ref, o_ref):
        agg = a0_ref[...] + a1_ref[...]
        acc = jnp.dot(agg, wr_ref[...], preferred_element_type=jnp.float32)
        acc += jnp.dot(x_ref[...], wo_ref[...], preferred_element_type=jnp.float32)
        o_ref[...] = jnp.maximum(acc + b_ref[...], 0.0)

    row_spec = pl.BlockSpec((blk, d), lambda i: (i, 0))
    full_spec = pl.BlockSpec((d, d), lambda i: (0, 0))
    bias_spec = pl.BlockSpec((1, d), lambda i: (0, 0))
    return pl.pallas_call(
        body,
        grid=grid,
        in_specs=[row_spec, row_spec, row_spec, full_spec, full_spec, bias_spec],
        out_specs=row_spec,
        out_shape=jax.ShapeDtypeStruct((n, d), jnp.float32),
    )(a0, a1, x, w_rel_t, w_root_t, b_row)


def kernel(x, edge_index, dropout, W1_rel, b1_rel, W1_root, W2_rel, b2_rel, W2_root):
    n, d = x.shape
    e = edge_index.shape[1]

    n0, n1 = N_CHUNKS_C0, N_CHUNKS_C1
    e_pad = NS * (n0 + n1) * CHUNK
    n_pad = 128 * (-(-(n + 1) // 128))
    assert n < 2**15 and e_pad >= e and n0 % 8 == 0 and n1 % 8 == 0

    packed = edge_index[0] | (edge_index[1] << 16)
    pad = e_pad - e
    if pad:
        packed = jnp.concatenate(
            [packed, jnp.full((pad,), n << 16, jnp.int32)])
    packed = packed.reshape(NS * (n0 + n1), CHUNK)
    zeros = jnp.zeros((n_pad, d), jnp.float32)

    sc_scatter = _sc_scatter_kernel(n_pad, n0, n1, d)

    a0, a1 = sc_scatter(x, packed, zeros)
    h = _tc_layer_kernel(a0[:n], a1[:n], x, W1_rel.T, W1_root.T,
                         b1_rel.reshape(1, d))
    a0, a1 = sc_scatter(h, packed, zeros)
    out = _tc_layer_kernel(a0[:n], a1[:n], h, W2_rel.T, W2_root.T,
                           b2_rel.reshape(1, d))
    return out
